# baseline (speedup 1.0000x reference)
import functools

import jax
import jax.numpy as jnp
from jax import lax
from jax.experimental import pallas as pl
from jax.experimental.pallas import tpu as pltpu
from jax.experimental.pallas import tpu_sc as plsc

_N = 10000
_E = 320000
_FIN = 128
_HID = 64
_CLS = 40
_CP = 48

_NCORE, _NSUB, _LANES = 2, 16, 16
_NW = _NCORE * _NSUB
_CHUNK = 128
_NCHUNK = 80
_EPT = _NCHUNK * _CHUNK
_EPAD = _NW * _EPT
_NGRP = _CHUNK // _LANES
_NPAD = 10240
_RPT = _NPAD // _NSUB
_ZROWS = 128

_MESH = dict(core_axis_name="c", subcore_axis_name="s")
_SC_PARAMS = pltpu.CompilerParams(
    use_tc_tiling_on_sc=False, needs_layout_passes=False
)


def _sc_deg(dst2, w2):
    @functools.partial(
        pl.kernel,
        mesh=plsc.VectorSubcoreMesh(**_MESH),
        out_type=jax.ShapeDtypeStruct((_NCORE, _NPAD, _LANES), jnp.float32),
        scratch_types=[
            pltpu.VMEM((_NCHUNK, _CHUNK), jnp.int32),
            pltpu.VMEM((_NCHUNK, _CHUNK), jnp.float32),
            pltpu.VMEM((_CHUNK, _LANES), jnp.float32),
            pltpu.VMEM((_CHUNK, _LANES), jnp.float32),
            pltpu.VMEM((_ZROWS, _LANES), jnp.float32),
            pltpu.VMEM_SHARED((_NPAD, _LANES), jnp.float32),
            pltpu.SemaphoreType.DMA,
            pltpu.SemaphoreType.DMA,
        ],
        compiler_params=_SC_PARAMS,
    )
    def k(dst_hbm, w_hbm, out_hbm, dst_v, w_v, vbuf0, vbuf1, zbuf, acc,
          ssem0, ssem1):
        c = lax.axis_index("c")
        s = lax.axis_index("s")
        wid = c * _NSUB + s
        zero = jnp.zeros((_LANES,), jnp.float32)
        vbuf = (vbuf0, vbuf1)
        ssem = (ssem0, ssem1)

        def zrow(i, carry):
            zbuf[i, :] = zero
            return carry

        lax.fori_loop(0, _ZROWS, zrow, 0)
        row0 = s * _RPT
        for r in range(_RPT // _ZROWS):
            pltpu.sync_copy(zbuf, acc.at[pl.ds(row0 + r * _ZROWS, _ZROWS)])
        pltpu.sync_copy(dst_hbm.at[wid], dst_v)
        pltpu.sync_copy(w_hbm.at[wid], w_v)
        plsc.subcore_barrier()

        def fill(ci, vb):
            def grp(g, carry2):
                wv = w_v[ci, pl.ds(g * _LANES, _LANES)]
                for kk in range(_LANES):
                    vb[g * _LANES + kk, :] = zero + wv[kk]
                return carry2

            lax.fori_loop(0, _NGRP, grp, 0)

        def scatter_start(ci, b):
            pltpu.async_copy(vbuf[b], acc.at[dst_v.at[ci]], ssem[b], add=True)

        def scatter_wait(ci, b):
            pltpu.make_async_copy(
                vbuf[b], acc.at[dst_v.at[ci]], ssem[b]
            ).wait()

        def pair(i, carry):
            for b in range(2):
                ci = 2 * i + b

                @pl.when(i > 0)
                def _():
                    scatter_wait(ci - 2, b)

                fill(ci, vbuf[b])
                scatter_start(ci, b)
            return carry

        lax.fori_loop(0, _NCHUNK // 2, pair, 0)
        scatter_wait(_NCHUNK - 2, 0)
        scatter_wait(_NCHUNK - 1, 1)
        plsc.subcore_barrier()
        for r in range(_RPT // _ZROWS):
            ro = row0 + r * _ZROWS
            pltpu.sync_copy(acc.at[pl.ds(ro, _ZROWS)], out_hbm.at[c, pl.ds(ro, _ZROWS)])

    return k(dst2, w2)


def _sc_agg(hs, src2, dst2, w2, d):
    nj = d // _LANES

    @functools.partial(
        pl.kernel,
        mesh=plsc.VectorSubcoreMesh(**_MESH),
        out_type=jax.ShapeDtypeStruct((_NCORE, _NPAD, d), jnp.float32),
        scratch_types=[
            pltpu.VMEM((_NCHUNK, _CHUNK), jnp.int32),
            pltpu.VMEM((_NCHUNK, _CHUNK), jnp.int32),
            pltpu.VMEM((_NCHUNK, _CHUNK), jnp.float32),
            pltpu.VMEM((_CHUNK, d), jnp.float32),
            pltpu.VMEM((_CHUNK, d), jnp.float32),
            pltpu.VMEM((_CHUNK, d), jnp.float32),
            pltpu.VMEM((_CHUNK, d), jnp.float32),
            pltpu.VMEM((_ZROWS, d), jnp.float32),
            pltpu.VMEM_SHARED((_NPAD, d), jnp.float32),
            pltpu.SemaphoreType.DMA,
            pltpu.SemaphoreType.DMA,
            pltpu.SemaphoreType.DMA,
            pltpu.SemaphoreType.DMA,
        ],
        compiler_params=_SC_PARAMS,
    )
    def k(hs_hbm, src_hbm, dst_hbm, w_hbm, out_hbm,
          src_v, dst_v, w_v, gbuf0, gbuf1, sbuf0, sbuf1, zbuf, acc,
          gsem0, gsem1, ssem0, ssem1):
        c = lax.axis_index("c")
        s = lax.axis_index("s")
        wid = c * _NSUB + s
        zero = jnp.zeros((_LANES,), jnp.float32)
        gbuf = (gbuf0, gbuf1)
        sbuf = (sbuf0, sbuf1)
        gsem = (gsem0, gsem1)
        ssem = (ssem0, ssem1)

        def zrow(i, carry):
            for j in range(nj):
                zbuf[i, pl.ds(j * _LANES, _LANES)] = zero
            return carry

        lax.fori_loop(0, _ZROWS, zrow, 0)
        row0 = s * _RPT
        for r in range(_RPT // _ZROWS):
            pltpu.sync_copy(zbuf, acc.at[pl.ds(row0 + r * _ZROWS, _ZROWS)])
        pltpu.sync_copy(src_hbm.at[wid], src_v)
        pltpu.sync_copy(dst_hbm.at[wid], dst_v)
        pltpu.sync_copy(w_hbm.at[wid], w_v)
        plsc.subcore_barrier()

        def mul_into(ci, gb, sb):
            def mul(g, carry2):
                wv = w_v[ci, pl.ds(g * _LANES, _LANES)]
                for kk in range(_LANES):
                    wk = wv[kk]
                    e = g * _LANES + kk
                    for j in range(nj):
                        sl = pl.ds(j * _LANES, _LANES)
                        sb[e, sl] = gb[e, sl] * wk
                return carry2

            lax.fori_loop(0, _NGRP, mul, 0)

        def gather_start(ci, b):
            pltpu.async_copy(hs_hbm.at[src_v.at[ci]], gbuf[b], gsem[b])

        def gather_wait(ci, b):
            pltpu.make_async_copy(
                hs_hbm.at[src_v.at[ci]], gbuf[b], gsem[b]
            ).wait()

        def scatter_start(ci, b):
            pltpu.async_copy(sbuf[b], acc.at[dst_v.at[ci]], ssem[b], add=True)

        def scatter_wait(ci, b):
            pltpu.make_async_copy(
                sbuf[b], acc.at[dst_v.at[ci]], ssem[b]
            ).wait()

        gather_start(0, 0)
        gather_start(1, 1)

        def pair(i, carry):
            for b in range(2):
                ci = 2 * i + b
                gather_wait(ci, b)

                @pl.when(i > 0)
                def _():
                    scatter_wait(ci - 2, b)

                mul_into(ci, gbuf[b], sbuf[b])

                @pl.when(ci + 2 < _NCHUNK)
                def _():
                    gather_start(ci + 2, b)

                scatter_start(ci, b)
            return carry

        lax.fori_loop(0, _NCHUNK // 2, pair, 0)
        scatter_wait(_NCHUNK - 2, 0)
        scatter_wait(_NCHUNK - 1, 1)
        plsc.subcore_barrier()
        for r in range(_RPT // _ZROWS):
            ro = row0 + r * _ZROWS
            pltpu.sync_copy(acc.at[pl.ds(ro, _ZROWS)], out_hbm.at[c, pl.ds(ro, _ZROWS)])

    return k(hs, src2, dst2, w2)


def _tc_layer1(degp, x, w1):
    def body(deg_ref, x_ref, w1_ref, hs_ref, dinv_ref):
        deg = deg_ref[0, :_N] + deg_ref[1, :_N] + 1.0
        dinv = lax.rsqrt(deg)[:, 0:1]
        h = jnp.dot(x_ref[...], w1_ref[...], preferred_element_type=jnp.float32)
        hs_ref[...] = h * dinv
        dinv_ref[...] = dinv

    return pl.pallas_call(
        body,
        out_shape=(
            jax.ShapeDtypeStruct((_N, _HID), jnp.float32),
            jax.ShapeDtypeStruct((_N, 1), jnp.float32),
        ),
    )(degp, x, w1)


def _tc_layer2(agg1, hs, dinv, b1, w2p):
    def body(a_ref, hs_ref, dinv_ref, b1_ref, w2_ref, out_ref):
        t = (a_ref[0, :_N] + a_ref[1, :_N] + hs_ref[...]) * dinv_ref[...] + b1_ref[...]
        r = jnp.maximum(t, 0.0)
        h2 = jnp.dot(r, w2_ref[...], preferred_element_type=jnp.float32)
        out_ref[...] = h2 * dinv_ref[...]

    return pl.pallas_call(
        body,
        out_shape=jax.ShapeDtypeStruct((_N, _CP), jnp.float32),
    )(agg1, hs, dinv, b1, w2p)


def _tc_out(agg2, hs2, dinv, b2):
    def body(a_ref, hs2_ref, dinv_ref, b2_ref, out_ref):
        t = (a_ref[0, :_N] + a_ref[1, :_N] + hs2_ref[...]) * dinv_ref[...]
        o = t[:, :_CLS] + b2_ref[...]
        m = jnp.max(o, axis=1, keepdims=True)
        z = o - m
        lse = jnp.log(jnp.sum(jnp.exp(z), axis=1, keepdims=True))
        out_ref[...] = z - lse

    return pl.pallas_call(
        body,
        out_shape=jax.ShapeDtypeStruct((_N, _CLS), jnp.float32),
    )(agg2, hs2, dinv, b2)


def kernel(x, edge_index, edge_weight, W1, b1, W2, b2):
    npad = _EPAD - _E
    src2 = jnp.pad(edge_index[0].astype(jnp.int32), (0, npad)).reshape(
        _NW, _NCHUNK, _CHUNK)
    dst2 = jnp.pad(edge_index[1].astype(jnp.int32), (0, npad)).reshape(
        _NW, _NCHUNK, _CHUNK)
    w2 = jnp.pad(edge_weight.astype(jnp.float32), (0, npad)).reshape(
        _NW, _NCHUNK, _CHUNK)
    x = x.astype(jnp.float32)

    degp = _sc_deg(dst2, w2)
    hs, dinv = _tc_layer1(degp, x, W1)
    agg1 = _sc_agg(hs, src2, dst2, w2, _HID)
    w2p = jnp.pad(W2, ((0, 0), (0, _CP - _CLS)))
    hs2 = _tc_layer2(agg1, hs, dinv, b1.reshape(1, _HID), w2p)
    agg2 = _sc_agg(hs2, src2, dst2, w2, _CP)
    return _tc_out(agg2, hs2, dinv, b2.reshape(1, _CLS))


# speedup vs baseline: 1.9306x; 1.9306x over previous
import functools

import jax
import jax.numpy as jnp
from jax import lax
from jax.experimental import pallas as pl
from jax.experimental.pallas import tpu as pltpu
from jax.experimental.pallas import tpu_sc as plsc

_N = 10000
_E = 320000
_FIN = 128
_HID = 64
_CLS = 40
_CP = 48

_NCORE, _NSUB, _LANES = 2, 16, 16
_NW = _NCORE * _NSUB
_CHUNK = 80
_NCHUNK = 125
_EPT = _NCHUNK * _CHUNK
_EPAD = _NW * _EPT
_NGRP = _CHUNK // _LANES
_NPAD = 10240
_RPT = _NPAD // _NSUB
_ZROWS = 128

_MESH = dict(core_axis_name="c", subcore_axis_name="s")
_SC_PARAMS = pltpu.CompilerParams(
    use_tc_tiling_on_sc=False, needs_layout_passes=False
)


def _sc_deg(dst2, w2):
    @functools.partial(
        pl.kernel,
        mesh=plsc.VectorSubcoreMesh(**_MESH),
        out_type=jax.ShapeDtypeStruct((_NCORE, _NPAD, _LANES), jnp.float32),
        scratch_types=[
            pltpu.VMEM((_NCHUNK, _CHUNK), jnp.int32),
            pltpu.VMEM((_NCHUNK, _CHUNK), jnp.float32),
            pltpu.VMEM((_CHUNK, _LANES), jnp.float32),
            pltpu.VMEM((_CHUNK, _LANES), jnp.float32),
            pltpu.VMEM((_ZROWS, _LANES), jnp.float32),
            pltpu.VMEM_SHARED((_NPAD, _LANES), jnp.float32),
            pltpu.SemaphoreType.DMA,
            pltpu.SemaphoreType.DMA,
        ],
        compiler_params=_SC_PARAMS,
    )
    def k(dst_hbm, w_hbm, out_hbm, dst_v, w_v, vbuf0, vbuf1, zbuf, acc,
          ssem0, ssem1):
        c = lax.axis_index("c")
        s = lax.axis_index("s")
        wid = c * _NSUB + s
        zero = jnp.zeros((_LANES,), jnp.float32)
        vbuf = (vbuf0, vbuf1)
        ssem = (ssem0, ssem1)

        def zrow(i, carry):
            zbuf[i, :] = zero
            return carry

        lax.fori_loop(0, _ZROWS, zrow, 0)
        row0 = s * _RPT
        for r in range(_RPT // _ZROWS):
            pltpu.sync_copy(zbuf, acc.at[pl.ds(row0 + r * _ZROWS, _ZROWS)])
        pltpu.sync_copy(dst_hbm.at[wid], dst_v)
        pltpu.sync_copy(w_hbm.at[wid], w_v)
        plsc.subcore_barrier()

        def fill(ci, vb):
            def grp(g, carry2):
                wv = w_v[ci, pl.ds(g * _LANES, _LANES)]
                for kk in range(_LANES):
                    vb[g * _LANES + kk, :] = zero + wv[kk]
                return carry2

            lax.fori_loop(0, _NGRP, grp, 0)

        def scatter_start(ci, b):
            pltpu.async_copy(vbuf[b], acc.at[dst_v.at[ci]], ssem[b], add=True)

        def scatter_wait(ci, b):
            pltpu.make_async_copy(
                vbuf[b], acc.at[dst_v.at[ci]], ssem[b]
            ).wait()

        def pair(i, carry):
            for b in range(2):
                ci = 2 * i + b

                @pl.when(i > 0)
                def _():
                    scatter_wait(ci - 2, b)

                fill(ci, vbuf[b])
                scatter_start(ci, b)
            return carry

        lax.fori_loop(0, _NCHUNK // 2, pair, 0)
        last = _NCHUNK - 1
        scatter_wait(last - 2, 0)
        fill(last, vbuf[0])
        scatter_start(last, 0)
        scatter_wait(last - 1, 1)
        scatter_wait(last, 0)
        plsc.subcore_barrier()
        for r in range(_RPT // _ZROWS):
            ro = row0 + r * _ZROWS
            pltpu.sync_copy(acc.at[pl.ds(ro, _ZROWS)], out_hbm.at[c, pl.ds(ro, _ZROWS)])

    return k(dst2, w2)


def _sc_agg(hs, src2, dst2, w2, d):
    nj = d // _LANES

    @functools.partial(
        pl.kernel,
        mesh=plsc.VectorSubcoreMesh(**_MESH),
        out_type=jax.ShapeDtypeStruct((_NCORE, _NPAD, d), jnp.float32),
        scratch_types=[
            pltpu.VMEM((_NCHUNK, _CHUNK), jnp.int32),
            pltpu.VMEM((_NCHUNK, _CHUNK), jnp.int32),
            pltpu.VMEM((_NCHUNK, _CHUNK), jnp.float32),
            pltpu.VMEM((_CHUNK, d), jnp.float32),
            pltpu.VMEM((_CHUNK, d), jnp.float32),
            pltpu.VMEM((_CHUNK, d), jnp.float32),
            pltpu.VMEM((_CHUNK, d), jnp.float32),
            pltpu.VMEM((_ZROWS, d), jnp.float32),
            pltpu.VMEM_SHARED((_NPAD, d), jnp.float32),
            pltpu.SemaphoreType.DMA,
            pltpu.SemaphoreType.DMA,
            pltpu.SemaphoreType.DMA,
            pltpu.SemaphoreType.DMA,
        ],
        compiler_params=_SC_PARAMS,
    )
    def k(hs_hbm, src_hbm, dst_hbm, w_hbm, out_hbm,
          src_v, dst_v, w_v, gbuf0, gbuf1, sbuf0, sbuf1, zbuf, acc,
          gsem0, gsem1, ssem0, ssem1):
        c = lax.axis_index("c")
        s = lax.axis_index("s")
        wid = c * _NSUB + s
        zero = jnp.zeros((_LANES,), jnp.float32)
        gbuf = (gbuf0, gbuf1)
        sbuf = (sbuf0, sbuf1)
        gsem = (gsem0, gsem1)
        ssem = (ssem0, ssem1)

        def zrow(i, carry):
            for j in range(nj):
                zbuf[i, pl.ds(j * _LANES, _LANES)] = zero
            return carry

        lax.fori_loop(0, _ZROWS, zrow, 0)
        row0 = s * _RPT
        for r in range(_RPT // _ZROWS):
            pltpu.sync_copy(zbuf, acc.at[pl.ds(row0 + r * _ZROWS, _ZROWS)])
        pltpu.sync_copy(src_hbm.at[wid], src_v)
        pltpu.sync_copy(dst_hbm.at[wid], dst_v)
        pltpu.sync_copy(w_hbm.at[wid], w_v)
        plsc.subcore_barrier()

        def mul_into(ci, gb, sb):
            def mul(g, carry2):
                wv = w_v[ci, pl.ds(g * _LANES, _LANES)]
                for kk in range(_LANES):
                    wk = wv[kk]
                    e = g * _LANES + kk
                    for j in range(nj):
                        sl = pl.ds(j * _LANES, _LANES)
                        sb[e, sl] = gb[e, sl] * wk
                return carry2

            lax.fori_loop(0, _NGRP, mul, 0)

        def gather_start(ci, b):
            pltpu.async_copy(hs_hbm.at[src_v.at[ci]], gbuf[b], gsem[b])

        def gather_wait(ci, b):
            pltpu.make_async_copy(
                hs_hbm.at[src_v.at[ci]], gbuf[b], gsem[b]
            ).wait()

        def scatter_start(ci, b):
            pltpu.async_copy(sbuf[b], acc.at[dst_v.at[ci]], ssem[b], add=True)

        def scatter_wait(ci, b):
            pltpu.make_async_copy(
                sbuf[b], acc.at[dst_v.at[ci]], ssem[b]
            ).wait()

        gather_start(0, 0)
        gather_start(1, 1)

        def pair(i, carry):
            for b in range(2):
                ci = 2 * i + b
                gather_wait(ci, b)

                @pl.when(i > 0)
                def _():
                    scatter_wait(ci - 2, b)

                mul_into(ci, gbuf[b], sbuf[b])

                @pl.when(ci + 2 < _NCHUNK)
                def _():
                    gather_start(ci + 2, b)

                scatter_start(ci, b)
            return carry

        lax.fori_loop(0, _NCHUNK // 2, pair, 0)
        last = _NCHUNK - 1
        gather_wait(last, 0)
        scatter_wait(last - 2, 0)
        mul_into(last, gbuf[0], sbuf[0])
        scatter_start(last, 0)
        scatter_wait(last - 1, 1)
        scatter_wait(last, 0)
        plsc.subcore_barrier()
        for r in range(_RPT // _ZROWS):
            ro = row0 + r * _ZROWS
            pltpu.sync_copy(acc.at[pl.ds(ro, _ZROWS)], out_hbm.at[c, pl.ds(ro, _ZROWS)])

    return k(hs, src2, dst2, w2)


def _tc_layer1(degp, x, w1):
    def body(deg_ref, x_ref, w1_ref, hs_ref, dinv_ref):
        deg = deg_ref[0, :_N] + deg_ref[1, :_N] + 1.0
        dinv = lax.rsqrt(deg)[:, 0:1]
        h = jnp.dot(x_ref[...], w1_ref[...], preferred_element_type=jnp.float32)
        hs_ref[...] = h * dinv
        dinv_ref[...] = dinv

    return pl.pallas_call(
        body,
        out_shape=(
            jax.ShapeDtypeStruct((_N, _HID), jnp.float32),
            jax.ShapeDtypeStruct((_N, 1), jnp.float32),
        ),
    )(degp, x, w1)


def _tc_layer2(agg1, hs, dinv, b1, w2p):
    def body(a_ref, hs_ref, dinv_ref, b1_ref, w2_ref, out_ref):
        t = (a_ref[0, :_N] + a_ref[1, :_N] + hs_ref[...]) * dinv_ref[...] + b1_ref[...]
        r = jnp.maximum(t, 0.0)
        h2 = jnp.dot(r, w2_ref[...], preferred_element_type=jnp.float32)
        out_ref[...] = h2 * dinv_ref[...]

    return pl.pallas_call(
        body,
        out_shape=jax.ShapeDtypeStruct((_N, _CP), jnp.float32),
    )(agg1, hs, dinv, b1, w2p)


def _tc_out(agg2, hs2, dinv, b2):
    def body(a_ref, hs2_ref, dinv_ref, b2_ref, out_ref):
        t = (a_ref[0, :_N] + a_ref[1, :_N] + hs2_ref[...]) * dinv_ref[...]
        o = t[:, :_CLS] + b2_ref[...]
        m = jnp.max(o, axis=1, keepdims=True)
        z = o - m
        lse = jnp.log(jnp.sum(jnp.exp(z), axis=1, keepdims=True))
        out_ref[...] = z - lse

    return pl.pallas_call(
        body,
        out_shape=jax.ShapeDtypeStruct((_N, _CLS), jnp.float32),
    )(agg2, hs2, dinv, b2)


def kernel(x, edge_index, edge_weight, W1, b1, W2, b2):
    npad = _EPAD - _E
    src2 = jnp.pad(edge_index[0].astype(jnp.int32), (0, npad)).reshape(
        _NW, _NCHUNK, _CHUNK)
    dst2 = jnp.pad(edge_index[1].astype(jnp.int32), (0, npad)).reshape(
        _NW, _NCHUNK, _CHUNK)
    w2 = jnp.pad(edge_weight.astype(jnp.float32), (0, npad)).reshape(
        _NW, _NCHUNK, _CHUNK)
    x = x.astype(jnp.float32)

    degp = _sc_deg(dst2, w2)
    hs, dinv = _tc_layer1(degp, x, W1)
    agg1 = _sc_agg(hs, src2, dst2, w2, _HID)
    w2p = jnp.pad(W2, ((0, 0), (0, _CP - _CLS)))
    hs2 = _tc_layer2(agg1, hs, dinv, b1.reshape(1, _HID), w2p)
    agg2 = _sc_agg(hs2, src2, dst2, w2, _CP)
    return _tc_out(agg2, hs2, dinv, b2.reshape(1, _CLS))


# speedup vs baseline: 1.9486x; 1.0093x over previous
import functools

import jax
import jax.numpy as jnp
from jax import lax
from jax.experimental import pallas as pl
from jax.experimental.pallas import tpu as pltpu
from jax.experimental.pallas import tpu_sc as plsc

_N = 10000
_E = 320000
_FIN = 128
_HID = 64
_CLS = 40
_CP = 48

_NCORE, _NSUB, _LANES = 2, 16, 16
_NW = _NCORE * _NSUB
_CHUNK = 80
_NCHUNK = 125
_EPT = _NCHUNK * _CHUNK
_EPAD = _NW * _EPT
_NGRP = _CHUNK // _LANES
_NPAD = 10240
_RPT = _NPAD // _NSUB
_ZROWS = 128

_MESH = dict(core_axis_name="c", subcore_axis_name="s")
_SC_PARAMS = pltpu.CompilerParams(
    use_tc_tiling_on_sc=False, needs_layout_passes=False
)


def _sc_deg(dst2, w2):
    @functools.partial(
        pl.kernel,
        mesh=plsc.VectorSubcoreMesh(**_MESH),
        out_type=jax.ShapeDtypeStruct((_NCORE, _NPAD, _LANES), jnp.float32),
        scratch_types=[
            pltpu.VMEM((_NCHUNK, _CHUNK), jnp.int32),
            pltpu.VMEM((_NCHUNK, _CHUNK), jnp.float32),
            pltpu.VMEM((_CHUNK, _LANES), jnp.float32),
            pltpu.VMEM((_CHUNK, _LANES), jnp.float32),
            pltpu.VMEM((_ZROWS, _LANES), jnp.float32),
            pltpu.VMEM_SHARED((_NPAD, _LANES), jnp.float32),
            pltpu.SemaphoreType.DMA,
            pltpu.SemaphoreType.DMA,
        ],
        compiler_params=_SC_PARAMS,
    )
    def k(dst_hbm, w_hbm, out_hbm, dst_v, w_v, vbuf0, vbuf1, zbuf, acc,
          ssem0, ssem1):
        c = lax.axis_index("c")
        s = lax.axis_index("s")
        wid = c * _NSUB + s
        zero = jnp.zeros((_LANES,), jnp.float32)
        vbuf = (vbuf0, vbuf1)
        ssem = (ssem0, ssem1)

        def zrow(i, carry):
            zbuf[i, :] = zero
            return carry

        lax.fori_loop(0, _ZROWS, zrow, 0)
        row0 = s * _RPT
        for r in range(_RPT // _ZROWS):
            pltpu.sync_copy(zbuf, acc.at[pl.ds(row0 + r * _ZROWS, _ZROWS)])
        pltpu.sync_copy(dst_hbm.at[wid], dst_v)
        pltpu.sync_copy(w_hbm.at[wid], w_v)
        plsc.subcore_barrier()

        def fill(ci, vb):
            @plsc.parallel_loop(0, _NGRP, unroll=2)
            def grp(g):
                wv = w_v[ci, pl.ds(g * _LANES, _LANES)]
                for kk in range(_LANES):
                    vb[g * _LANES + kk, :] = zero + wv[kk]

        def scatter_start(ci, b):
            pltpu.async_copy(vbuf[b], acc.at[dst_v.at[ci]], ssem[b], add=True)

        def scatter_wait(ci, b):
            pltpu.make_async_copy(
                vbuf[b], acc.at[dst_v.at[ci]], ssem[b]
            ).wait()

        def pair(i, carry):
            for b in range(2):
                ci = 2 * i + b

                @pl.when(i > 0)
                def _():
                    scatter_wait(ci - 2, b)

                fill(ci, vbuf[b])
                scatter_start(ci, b)
            return carry

        lax.fori_loop(0, _NCHUNK // 2, pair, 0)
        last = _NCHUNK - 1
        scatter_wait(last - 2, 0)
        fill(last, vbuf[0])
        scatter_start(last, 0)
        scatter_wait(last - 1, 1)
        scatter_wait(last, 0)
        plsc.subcore_barrier()
        for r in range(_RPT // _ZROWS):
            ro = row0 + r * _ZROWS
            pltpu.sync_copy(acc.at[pl.ds(ro, _ZROWS)], out_hbm.at[c, pl.ds(ro, _ZROWS)])

    return k(dst2, w2)


def _sc_agg(hs, src2, dst2, w2, d):
    nj = d // _LANES

    @functools.partial(
        pl.kernel,
        mesh=plsc.VectorSubcoreMesh(**_MESH),
        out_type=jax.ShapeDtypeStruct((_NCORE, _NPAD, d), jnp.float32),
        scratch_types=[
            pltpu.VMEM((_NCHUNK, _CHUNK), jnp.int32),
            pltpu.VMEM((_NCHUNK, _CHUNK), jnp.int32),
            pltpu.VMEM((_NCHUNK, _CHUNK), jnp.float32),
            pltpu.VMEM((_CHUNK, d), jnp.float32),
            pltpu.VMEM((_CHUNK, d), jnp.float32),
            pltpu.VMEM((_CHUNK, d), jnp.float32),
            pltpu.VMEM((_CHUNK, d), jnp.float32),
            pltpu.VMEM((_ZROWS, d), jnp.float32),
            pltpu.VMEM_SHARED((_NPAD, d), jnp.float32),
            pltpu.SemaphoreType.DMA,
            pltpu.SemaphoreType.DMA,
            pltpu.SemaphoreType.DMA,
            pltpu.SemaphoreType.DMA,
        ],
        compiler_params=_SC_PARAMS,
    )
    def k(hs_hbm, src_hbm, dst_hbm, w_hbm, out_hbm,
          src_v, dst_v, w_v, gbuf0, gbuf1, sbuf0, sbuf1, zbuf, acc,
          gsem0, gsem1, ssem0, ssem1):
        c = lax.axis_index("c")
        s = lax.axis_index("s")
        wid = c * _NSUB + s
        zero = jnp.zeros((_LANES,), jnp.float32)
        gbuf = (gbuf0, gbuf1)
        sbuf = (sbuf0, sbuf1)
        gsem = (gsem0, gsem1)
        ssem = (ssem0, ssem1)

        def zrow(i, carry):
            for j in range(nj):
                zbuf[i, pl.ds(j * _LANES, _LANES)] = zero
            return carry

        lax.fori_loop(0, _ZROWS, zrow, 0)
        row0 = s * _RPT
        for r in range(_RPT // _ZROWS):
            pltpu.sync_copy(zbuf, acc.at[pl.ds(row0 + r * _ZROWS, _ZROWS)])
        pltpu.sync_copy(src_hbm.at[wid], src_v)
        pltpu.sync_copy(dst_hbm.at[wid], dst_v)
        pltpu.sync_copy(w_hbm.at[wid], w_v)
        plsc.subcore_barrier()

        def mul_into(ci, gb, sb):
            @plsc.parallel_loop(0, _NGRP, unroll=2)
            def mul(g):
                wv = w_v[ci, pl.ds(g * _LANES, _LANES)]
                for kk in range(_LANES):
                    wk = wv[kk]
                    e = g * _LANES + kk
                    for j in range(nj):
                        sl = pl.ds(j * _LANES, _LANES)
                        sb[e, sl] = gb[e, sl] * wk

        def gather_start(ci, b):
            pltpu.async_copy(hs_hbm.at[src_v.at[ci]], gbuf[b], gsem[b])

        def gather_wait(ci, b):
            pltpu.make_async_copy(
                hs_hbm.at[src_v.at[ci]], gbuf[b], gsem[b]
            ).wait()

        def scatter_start(ci, b):
            pltpu.async_copy(sbuf[b], acc.at[dst_v.at[ci]], ssem[b], add=True)

        def scatter_wait(ci, b):
            pltpu.make_async_copy(
                sbuf[b], acc.at[dst_v.at[ci]], ssem[b]
            ).wait()

        gather_start(0, 0)
        gather_start(1, 1)

        def pair(i, carry):
            for b in range(2):
                ci = 2 * i + b
                gather_wait(ci, b)

                @pl.when(i > 0)
                def _():
                    scatter_wait(ci - 2, b)

                mul_into(ci, gbuf[b], sbuf[b])

                @pl.when(ci + 2 < _NCHUNK)
                def _():
                    gather_start(ci + 2, b)

                scatter_start(ci, b)
            return carry

        lax.fori_loop(0, _NCHUNK // 2, pair, 0)
        last = _NCHUNK - 1
        gather_wait(last, 0)
        scatter_wait(last - 2, 0)
        mul_into(last, gbuf[0], sbuf[0])
        scatter_start(last, 0)
        scatter_wait(last - 1, 1)
        scatter_wait(last, 0)
        plsc.subcore_barrier()
        for r in range(_RPT // _ZROWS):
            ro = row0 + r * _ZROWS
            pltpu.sync_copy(acc.at[pl.ds(ro, _ZROWS)], out_hbm.at[c, pl.ds(ro, _ZROWS)])

    return k(hs, src2, dst2, w2)


def _tc_layer1(degp, x, w1):
    def body(deg_ref, x_ref, w1_ref, hs_ref, dinv_ref):
        deg = deg_ref[0, :_N] + deg_ref[1, :_N] + 1.0
        dinv = lax.rsqrt(deg)[:, 0:1]
        h = jnp.dot(x_ref[...], w1_ref[...], preferred_element_type=jnp.float32)
        hs_ref[...] = h * dinv
        dinv_ref[...] = dinv

    return pl.pallas_call(
        body,
        out_shape=(
            jax.ShapeDtypeStruct((_N, _HID), jnp.float32),
            jax.ShapeDtypeStruct((_N, 1), jnp.float32),
        ),
    )(degp, x, w1)


def _tc_layer2(agg1, hs, dinv, b1, w2p):
    def body(a_ref, hs_ref, dinv_ref, b1_ref, w2_ref, out_ref):
        t = (a_ref[0, :_N] + a_ref[1, :_N] + hs_ref[...]) * dinv_ref[...] + b1_ref[...]
        r = jnp.maximum(t, 0.0)
        h2 = jnp.dot(r, w2_ref[...], preferred_element_type=jnp.float32)
        out_ref[...] = h2 * dinv_ref[...]

    return pl.pallas_call(
        body,
        out_shape=jax.ShapeDtypeStruct((_N, _CP), jnp.float32),
    )(agg1, hs, dinv, b1, w2p)


def _tc_out(agg2, hs2, dinv, b2):
    def body(a_ref, hs2_ref, dinv_ref, b2_ref, out_ref):
        t = (a_ref[0, :_N] + a_ref[1, :_N] + hs2_ref[...]) * dinv_ref[...]
        o = t[:, :_CLS] + b2_ref[...]
        m = jnp.max(o, axis=1, keepdims=True)
        z = o - m
        lse = jnp.log(jnp.sum(jnp.exp(z), axis=1, keepdims=True))
        out_ref[...] = z - lse

    return pl.pallas_call(
        body,
        out_shape=jax.ShapeDtypeStruct((_N, _CLS), jnp.float32),
    )(agg2, hs2, dinv, b2)


def kernel(x, edge_index, edge_weight, W1, b1, W2, b2):
    npad = _EPAD - _E
    src2 = jnp.pad(edge_index[0].astype(jnp.int32), (0, npad)).reshape(
        _NW, _NCHUNK, _CHUNK)
    dst2 = jnp.pad(edge_index[1].astype(jnp.int32), (0, npad)).reshape(
        _NW, _NCHUNK, _CHUNK)
    w2 = jnp.pad(edge_weight.astype(jnp.float32), (0, npad)).reshape(
        _NW, _NCHUNK, _CHUNK)
    x = x.astype(jnp.float32)

    degp = _sc_deg(dst2, w2)
    hs, dinv = _tc_layer1(degp, x, W1)
    agg1 = _sc_agg(hs, src2, dst2, w2, _HID)
    w2p = jnp.pad(W2, ((0, 0), (0, _CP - _CLS)))
    hs2 = _tc_layer2(agg1, hs, dinv, b1.reshape(1, _HID), w2p)
    agg2 = _sc_agg(hs2, src2, dst2, w2, _CP)
    return _tc_out(agg2, hs2, dinv, b2.reshape(1, _CLS))


# speedup vs baseline: 2.2766x; 1.1683x over previous
import functools

import jax
import jax.numpy as jnp
from jax import lax
from jax.experimental import pallas as pl
from jax.experimental.pallas import tpu as pltpu
from jax.experimental.pallas import tpu_sc as plsc

_N = 10000
_E = 320000
_FIN = 128
_HID = 64
_CLS = 40
_CP = 48

_NCORE, _NSUB, _LANES = 2, 16, 16
_NW = _NCORE * _NSUB
_CHUNK = 80
_NCHUNK = 125
_EPT = _NCHUNK * _CHUNK
_EPAD = _NW * _EPT
_NGRP = _CHUNK // _LANES
_NPAD = 10240
_RPT = _NPAD // _NSUB
_ZROWS = 128

_MESH = dict(core_axis_name="c", subcore_axis_name="s")
_SC_PARAMS = pltpu.CompilerParams(
    use_tc_tiling_on_sc=False, needs_layout_passes=False
)


def _sc_deg(dst2, w2):
    @functools.partial(
        pl.kernel,
        mesh=plsc.VectorSubcoreMesh(**_MESH),
        out_type=jax.ShapeDtypeStruct((_NCORE, _NPAD, _LANES), jnp.float32),
        scratch_types=[
            pltpu.VMEM((_NCHUNK, _CHUNK), jnp.int32),
            pltpu.VMEM((_NCHUNK, _CHUNK), jnp.float32),
            pltpu.VMEM((_CHUNK, _LANES), jnp.float32),
            pltpu.VMEM((_CHUNK, _LANES), jnp.float32),
            pltpu.VMEM((_ZROWS, _LANES), jnp.float32),
            pltpu.VMEM_SHARED((_NPAD, _LANES), jnp.float32),
            pltpu.SemaphoreType.DMA,
            pltpu.SemaphoreType.DMA,
        ],
        compiler_params=_SC_PARAMS,
    )
    def k(dst_hbm, w_hbm, out_hbm, dst_v, w_v, vbuf0, vbuf1, zbuf, acc,
          ssem0, ssem1):
        c = lax.axis_index("c")
        s = lax.axis_index("s")
        wid = c * _NSUB + s
        zero = jnp.zeros((_LANES,), jnp.float32)
        vbuf = (vbuf0, vbuf1)
        ssem = (ssem0, ssem1)

        def zrow(i, carry):
            zbuf[i, :] = zero
            return carry

        lax.fori_loop(0, _ZROWS, zrow, 0)
        row0 = s * _RPT
        for r in range(_RPT // _ZROWS):
            pltpu.sync_copy(zbuf, acc.at[pl.ds(row0 + r * _ZROWS, _ZROWS)])
        pltpu.sync_copy(dst_hbm.at[wid], dst_v)
        pltpu.sync_copy(w_hbm.at[wid], w_v)
        plsc.subcore_barrier()

        def fill(ci, vb):
            @plsc.parallel_loop(0, _NGRP, unroll=2)
            def grp(g):
                wv = w_v[ci, pl.ds(g * _LANES, _LANES)]
                for kk in range(_LANES):
                    vb[g * _LANES + kk, :] = zero + wv[kk]

        def scatter_start(ci, b):
            pltpu.async_copy(vbuf[b], acc.at[dst_v.at[ci]], ssem[b], add=True)

        def scatter_wait(ci, b):
            pltpu.make_async_copy(
                vbuf[b], acc.at[dst_v.at[ci]], ssem[b]
            ).wait()

        def pair(i, carry):
            for b in range(2):
                ci = 2 * i + b

                @pl.when(i > 0)
                def _():
                    scatter_wait(ci - 2, b)

                fill(ci, vbuf[b])
                scatter_start(ci, b)
            return carry

        lax.fori_loop(0, _NCHUNK // 2, pair, 0)
        last = _NCHUNK - 1
        scatter_wait(last - 2, 0)
        fill(last, vbuf[0])
        scatter_start(last, 0)
        scatter_wait(last - 1, 1)
        scatter_wait(last, 0)
        plsc.subcore_barrier()
        for r in range(_RPT // _ZROWS):
            ro = row0 + r * _ZROWS
            pltpu.sync_copy(acc.at[pl.ds(ro, _ZROWS)], out_hbm.at[c, pl.ds(ro, _ZROWS)])

    return k(dst2, w2)


def _sc_agg(hs, src2, dst2, w2, d):
    nj = d // _LANES

    @functools.partial(
        pl.kernel,
        mesh=plsc.VectorSubcoreMesh(**_MESH),
        out_type=jax.ShapeDtypeStruct((_NCORE, _NPAD, d), jnp.float32),
        scratch_types=[
            pltpu.VMEM((_NCHUNK, _CHUNK), jnp.int32),
            pltpu.VMEM((_NCHUNK, _CHUNK), jnp.int32),
            pltpu.VMEM((_NCHUNK, _CHUNK), jnp.float32),
            pltpu.VMEM((_CHUNK, d), jnp.float32),
            pltpu.VMEM((_CHUNK, d), jnp.float32),
            pltpu.VMEM((_CHUNK, d), jnp.float32),
            pltpu.VMEM((_CHUNK, d), jnp.float32),
            pltpu.VMEM((_CHUNK, d), jnp.float32),
            pltpu.VMEM((_CHUNK, d), jnp.float32),
            pltpu.VMEM((_CHUNK, d), jnp.float32),
            pltpu.VMEM((_CHUNK, d), jnp.float32),
            pltpu.VMEM((_ZROWS, d), jnp.float32),
            pltpu.VMEM_SHARED((_NPAD, d), jnp.float32),
            pltpu.SemaphoreType.DMA,
            pltpu.SemaphoreType.DMA,
            pltpu.SemaphoreType.DMA,
            pltpu.SemaphoreType.DMA,
            pltpu.SemaphoreType.DMA,
            pltpu.SemaphoreType.DMA,
            pltpu.SemaphoreType.DMA,
            pltpu.SemaphoreType.DMA,
        ],
        compiler_params=_SC_PARAMS,
    )
    def k(hs_hbm, src_hbm, dst_hbm, w_hbm, out_hbm,
          src_v, dst_v, w_v, gbuf0, gbuf1, gbuf2, gbuf3,
          sbuf0, sbuf1, sbuf2, sbuf3, zbuf, acc,
          gsem0, gsem1, gsem2, gsem3, ssem0, ssem1, ssem2, ssem3):
        c = lax.axis_index("c")
        s = lax.axis_index("s")
        wid = c * _NSUB + s
        zero = jnp.zeros((_LANES,), jnp.float32)
        gbuf = (gbuf0, gbuf1, gbuf2, gbuf3)
        sbuf = (sbuf0, sbuf1, sbuf2, sbuf3)
        gsem = (gsem0, gsem1, gsem2, gsem3)
        ssem = (ssem0, ssem1, ssem2, ssem3)

        def zrow(i, carry):
            for j in range(nj):
                zbuf[i, pl.ds(j * _LANES, _LANES)] = zero
            return carry

        lax.fori_loop(0, _ZROWS, zrow, 0)
        row0 = s * _RPT
        for r in range(_RPT // _ZROWS):
            pltpu.sync_copy(zbuf, acc.at[pl.ds(row0 + r * _ZROWS, _ZROWS)])
        pltpu.sync_copy(src_hbm.at[wid], src_v)
        pltpu.sync_copy(dst_hbm.at[wid], dst_v)
        pltpu.sync_copy(w_hbm.at[wid], w_v)
        plsc.subcore_barrier()

        def mul_into(ci, gb, sb):
            @plsc.parallel_loop(0, _NGRP, unroll=2)
            def mul(g):
                wv = w_v[ci, pl.ds(g * _LANES, _LANES)]
                for kk in range(_LANES):
                    wk = wv[kk]
                    e = g * _LANES + kk
                    for j in range(nj):
                        sl = pl.ds(j * _LANES, _LANES)
                        sb[e, sl] = gb[e, sl] * wk

        def gather_start(ci, b):
            pltpu.async_copy(hs_hbm.at[src_v.at[ci]], gbuf[b], gsem[b])

        def gather_wait(ci, b):
            pltpu.make_async_copy(
                hs_hbm.at[src_v.at[ci]], gbuf[b], gsem[b]
            ).wait()

        def scatter_start(ci, b):
            pltpu.async_copy(sbuf[b], acc.at[dst_v.at[ci]], ssem[b], add=True)

        def scatter_wait(ci, b):
            pltpu.make_async_copy(
                sbuf[b], acc.at[dst_v.at[ci]], ssem[b]
            ).wait()

        nbuf = 4
        for b in range(nbuf):
            gather_start(b, b)

        def quad(i, carry):
            for b in range(nbuf):
                ci = nbuf * i + b
                gather_wait(ci, b)

                @pl.when(i > 0)
                def _():
                    scatter_wait(ci - nbuf, b)

                mul_into(ci, gbuf[b], sbuf[b])

                @pl.when(ci + nbuf < _NCHUNK)
                def _():
                    gather_start(ci + nbuf, b)

                scatter_start(ci, b)
            return carry

        lax.fori_loop(0, _NCHUNK // nbuf, quad, 0)
        last = _NCHUNK - 1
        gather_wait(last, 0)
        scatter_wait(last - nbuf, 0)
        mul_into(last, gbuf[0], sbuf[0])
        scatter_start(last, 0)
        for b in range(1, nbuf):
            scatter_wait(last - nbuf + b, b)
        scatter_wait(last, 0)
        plsc.subcore_barrier()
        for r in range(_RPT // _ZROWS):
            ro = row0 + r * _ZROWS
            pltpu.sync_copy(acc.at[pl.ds(ro, _ZROWS)], out_hbm.at[c, pl.ds(ro, _ZROWS)])

    return k(hs, src2, dst2, w2)


def _tc_layer1(degp, x, w1):
    def body(deg_ref, x_ref, w1_ref, hs_ref, dinv_ref):
        deg = deg_ref[0, :_N] + deg_ref[1, :_N] + 1.0
        dinv = lax.rsqrt(deg)[:, 0:1]
        h = jnp.dot(x_ref[...], w1_ref[...], preferred_element_type=jnp.float32)
        hs_ref[...] = h * dinv
        dinv_ref[...] = dinv

    return pl.pallas_call(
        body,
        out_shape=(
            jax.ShapeDtypeStruct((_N, _HID), jnp.float32),
            jax.ShapeDtypeStruct((_N, 1), jnp.float32),
        ),
    )(degp, x, w1)


def _tc_layer2(agg1, hs, dinv, b1, w2p):
    def body(a_ref, hs_ref, dinv_ref, b1_ref, w2_ref, out_ref):
        t = (a_ref[0, :_N] + a_ref[1, :_N] + hs_ref[...]) * dinv_ref[...] + b1_ref[...]
        r = jnp.maximum(t, 0.0)
        h2 = jnp.dot(r, w2_ref[...], preferred_element_type=jnp.float32)
        out_ref[...] = h2 * dinv_ref[...]

    return pl.pallas_call(
        body,
        out_shape=jax.ShapeDtypeStruct((_N, _CP), jnp.float32),
    )(agg1, hs, dinv, b1, w2p)


def _tc_out(agg2, hs2, dinv, b2):
    def body(a_ref, hs2_ref, dinv_ref, b2_ref, out_ref):
        t = (a_ref[0, :_N] + a_ref[1, :_N] + hs2_ref[...]) * dinv_ref[...]
        o = t[:, :_CLS] + b2_ref[...]
        m = jnp.max(o, axis=1, keepdims=True)
        z = o - m
        lse = jnp.log(jnp.sum(jnp.exp(z), axis=1, keepdims=True))
        out_ref[...] = z - lse

    return pl.pallas_call(
        body,
        out_shape=jax.ShapeDtypeStruct((_N, _CLS), jnp.float32),
    )(agg2, hs2, dinv, b2)


def kernel(x, edge_index, edge_weight, W1, b1, W2, b2):
    npad = _EPAD - _E
    src2 = jnp.pad(edge_index[0].astype(jnp.int32), (0, npad)).reshape(
        _NW, _NCHUNK, _CHUNK)
    dst2 = jnp.pad(edge_index[1].astype(jnp.int32), (0, npad)).reshape(
        _NW, _NCHUNK, _CHUNK)
    w2 = jnp.pad(edge_weight.astype(jnp.float32), (0, npad)).reshape(
        _NW, _NCHUNK, _CHUNK)
    x = x.astype(jnp.float32)

    degp = _sc_deg(dst2, w2)
    hs, dinv = _tc_layer1(degp, x, W1)
    agg1 = _sc_agg(hs, src2, dst2, w2, _HID)
    w2p = jnp.pad(W2, ((0, 0), (0, _CP - _CLS)))
    hs2 = _tc_layer2(agg1, hs, dinv, b1.reshape(1, _HID), w2p)
    agg2 = _sc_agg(hs2, src2, dst2, w2, _CP)
    return _tc_out(agg2, hs2, dinv, b2.reshape(1, _CLS))
